# baseline (speedup 1.0000x reference)
import jax
import jax.numpy as jnp
from jax.experimental import pallas as pl
from jax.experimental.pallas import tpu as pltpu

_NT = (((1,), (1,)), ((), ()))
_OUTER = (((0,), (0,)), ((), ()))
_GPB = 2


def _sage_kernel(adj_ref, x_ref, mask_ref, w1_ref, b1_ref, w2_ref, b2_ref,
                 wfc_ref, bfc_ref, out_ref,
                 a_buf, xp0t_buf, h1t_buf, xp1t_buf):
    s = pl.program_id(0)
    npairs = pl.num_programs(0) - 1
    par = jax.lax.rem(s, 2)
    parb = jax.lax.rem(s + 1, 2)
    n = adj_ref.shape[1]
    gs = range(_GPB)
    ones_row = jnp.ones((1, n), jnp.float32)

    sf = jnp.minimum(s, npairs - 1)
    sb = jnp.maximum(s, 1) - 1

    b1c = jax.lax.dot_general(b1_ref[...], ones_row, _OUTER,
                              preferred_element_type=jnp.float32)
    b2c = jax.lax.dot_general(b2_ref[...], ones_row, _OUTER,
                              preferred_element_type=jnp.float32)
    w1b = w1_ref[...].astype(jnp.bfloat16)
    w2b = w2_ref[...].astype(jnp.bfloat16)

    for g in gs:
        xp0t = jax.lax.dot_general(w1b, x_ref[g].astype(jnp.bfloat16), _NT,
                                   preferred_element_type=jnp.float32)
        xp0t_buf[g, 0:64, :] = xp0t.astype(jnp.bfloat16)
        xp0t_buf[g, 64:72, :] = jnp.ones((8, n), jnp.bfloat16)
        a_buf[par, g] = (adj_ref[g] > 0.98).astype(jnp.bfloat16)

    combs = [jax.lax.dot_general(xp0t_buf[g], a_buf[par, g], _NT,
                                 preferred_element_type=jnp.float32)
             for g in gs]
    for g in gs:
        m = mask_ref[pl.ds(sf * _GPB + g, 1), :]
        comb = combs[g] + xp0t_buf[g].astype(jnp.float32)
        inv = 1.0 / comb[64:65, :]
        h1t_buf[par, g] = (jnp.maximum(comb[0:64, :] * inv + b1c, 0.0)
                           * m).astype(jnp.bfloat16)

    for g in gs:
        xp1t = jnp.dot(w2b, h1t_buf[parb, g],
                       preferred_element_type=jnp.float32)
        xp1t_buf[g, 0:32, :] = xp1t.astype(jnp.bfloat16)
        xp1t_buf[g, 32:40, :] = jnp.ones((8, n), jnp.bfloat16)

    combs2 = [jax.lax.dot_general(xp1t_buf[g], a_buf[parb, g], _NT,
                                  preferred_element_type=jnp.float32)
              for g in gs]
    for g in gs:
        m = mask_ref[pl.ds(sb * _GPB + g, 1), :]
        comb2 = combs2[g] + xp1t_buf[g].astype(jnp.float32)
        inv2 = 1.0 / comb2[32:33, :]
        h2 = jnp.maximum(comb2[0:32, :] * inv2 + b2c, 0.0) * m
        pooled = jnp.max(h2, axis=1, keepdims=True)
        outrow = jax.lax.dot_general(pooled, wfc_ref[...],
                                     (((0,), (1,)), ((), ())),
                                     preferred_element_type=jnp.float32) \
            + bfc_ref[...]
        out_ref[pl.ds(sb * _GPB + g, 1), :] = outrow


def kernel(x, adj, mask, W1, b1, W2, b2, Wfc, bfc):
    B, N, F = x.shape
    H1 = W1.shape[0]
    H2 = W2.shape[0]
    OUT = Wfc.shape[0]
    npairs = B // _GPB

    return pl.pallas_call(
        _sage_kernel,
        grid=(npairs + 1,),
        in_specs=[
            pl.BlockSpec((_GPB, N, N),
                         lambda s: (jnp.minimum(s, npairs - 1), 0, 0)),
            pl.BlockSpec((_GPB, N, F),
                         lambda s: (jnp.minimum(s, npairs - 1), 0, 0)),
            pl.BlockSpec((B, N), lambda s: (0, 0)),
            pl.BlockSpec((H1, F), lambda s: (0, 0)),
            pl.BlockSpec((1, H1), lambda s: (0, 0)),
            pl.BlockSpec((H2, H1), lambda s: (0, 0)),
            pl.BlockSpec((1, H2), lambda s: (0, 0)),
            pl.BlockSpec((OUT, H2), lambda s: (0, 0)),
            pl.BlockSpec((1, OUT), lambda s: (0, 0)),
        ],
        out_specs=pl.BlockSpec((B, OUT), lambda s: (0, 0)),
        out_shape=jax.ShapeDtypeStruct((B, OUT), jnp.float32),
        scratch_shapes=[
            pltpu.VMEM((2, _GPB, N, N), jnp.bfloat16),
            pltpu.VMEM((_GPB, 72, N), jnp.bfloat16),
            pltpu.VMEM((2, _GPB, 64, N), jnp.bfloat16),
            pltpu.VMEM((_GPB, 40, N), jnp.bfloat16),
        ],
    )(adj, x, mask, W1, b1.reshape(1, H1), W2, b2.reshape(1, H2),
      Wfc, bfc.reshape(1, OUT))


# speedup vs baseline: 1.2334x; 1.2334x over previous
import jax
import jax.numpy as jnp
from jax.experimental import pallas as pl
from jax.experimental.pallas import tpu as pltpu

_NT = (((1,), (1,)), ((), ()))
_OUTER = (((0,), (0,)), ((), ()))
_GPB = 2


def _sage_kernel(adj_hi_ref, adj_lo_ref, x_ref, mask_ref, w1_ref, b1_ref,
                 w2_ref, b2_ref, wfc_ref, bfc_ref, out_ref,
                 a_buf, xp0t_buf, h1t_buf, xp1t_buf):
    s = pl.program_id(0)
    n = a_buf.shape[2]
    nh = n // 2
    gs = range(_GPB)
    ones_row = jnp.ones((1, n), jnp.float32)
    ms = [mask_ref[pl.ds(s * _GPB + g, 1), :] for g in gs]

    b1c = jax.lax.dot_general(b1_ref[...], ones_row, _OUTER,
                              preferred_element_type=jnp.float32)
    b2c = jax.lax.dot_general(b2_ref[...], ones_row, _OUTER,
                              preferred_element_type=jnp.float32)
    w1b = w1_ref[...].astype(jnp.bfloat16)
    w2b = w2_ref[...].astype(jnp.bfloat16)

    for g in gs:
        xp0t = jax.lax.dot_general(w1b, x_ref[g].astype(jnp.bfloat16), _NT,
                                   preferred_element_type=jnp.float32)
        xp0t_buf[g, 0:64, :] = xp0t.astype(jnp.bfloat16)
        xp0t_buf[g, 64:72, :] = jnp.ones((8, n), jnp.bfloat16)
        a_buf[g, 0:nh, :] = (adj_hi_ref[g] > 0.98).astype(jnp.bfloat16)
        a_buf[g, nh:n, :] = (adj_lo_ref[g] > 0.98).astype(jnp.bfloat16)

    combs = [jax.lax.dot_general(xp0t_buf[g], a_buf[g], _NT,
                                 preferred_element_type=jnp.float32)
             for g in gs]
    for g in gs:
        comb = combs[g] + xp0t_buf[g].astype(jnp.float32)
        inv = 1.0 / comb[64:65, :]
        h1t_buf[g] = (jnp.maximum(comb[0:64, :] * inv + b1c, 0.0)
                      * ms[g]).astype(jnp.bfloat16)

    for g in gs:
        xp1t = jnp.dot(w2b, h1t_buf[g], preferred_element_type=jnp.float32)
        xp1t_buf[g, 0:32, :] = xp1t.astype(jnp.bfloat16)
        xp1t_buf[g, 32:40, :] = jnp.ones((8, n), jnp.bfloat16)

    combs2 = [jax.lax.dot_general(xp1t_buf[g], a_buf[g], _NT,
                                  preferred_element_type=jnp.float32)
              for g in gs]
    for g in gs:
        comb2 = combs2[g] + xp1t_buf[g].astype(jnp.float32)
        inv2 = 1.0 / comb2[32:33, :]
        h2 = jnp.maximum(comb2[0:32, :] * inv2 + b2c, 0.0) * ms[g]
        pooled = jnp.max(h2, axis=1, keepdims=True)
        outrow = jax.lax.dot_general(pooled, wfc_ref[...],
                                     (((0,), (1,)), ((), ())),
                                     preferred_element_type=jnp.float32) \
            + bfc_ref[...]
        out_ref[pl.ds(s * _GPB + g, 1), :] = outrow


def kernel(x, adj, mask, W1, b1, W2, b2, Wfc, bfc):
    B, N, F = x.shape
    H1 = W1.shape[0]
    H2 = W2.shape[0]
    OUT = Wfc.shape[0]

    return pl.pallas_call(
        _sage_kernel,
        grid=(B // _GPB,),
        in_specs=[
            pl.BlockSpec((_GPB, N // 2, N), lambda s: (s, 0, 0)),
            pl.BlockSpec((_GPB, N // 2, N), lambda s: (s, 1, 0)),
            pl.BlockSpec((_GPB, N, F), lambda s: (s, 0, 0)),
            pl.BlockSpec((B, N), lambda s: (0, 0)),
            pl.BlockSpec((H1, F), lambda s: (0, 0)),
            pl.BlockSpec((1, H1), lambda s: (0, 0)),
            pl.BlockSpec((H2, H1), lambda s: (0, 0)),
            pl.BlockSpec((1, H2), lambda s: (0, 0)),
            pl.BlockSpec((OUT, H2), lambda s: (0, 0)),
            pl.BlockSpec((1, OUT), lambda s: (0, 0)),
        ],
        out_specs=pl.BlockSpec((B, OUT), lambda s: (0, 0)),
        out_shape=jax.ShapeDtypeStruct((B, OUT), jnp.float32),
        scratch_shapes=[
            pltpu.VMEM((_GPB, N, N), jnp.bfloat16),
            pltpu.VMEM((_GPB, 72, N), jnp.bfloat16),
            pltpu.VMEM((_GPB, 64, N), jnp.bfloat16),
            pltpu.VMEM((_GPB, 40, N), jnp.bfloat16),
        ],
    )(adj, adj, x, mask, W1, b1.reshape(1, H1), W2, b2.reshape(1, H2),
      Wfc, bfc.reshape(1, OUT))
